# baseline (speedup 1.0000x reference)
import functools

import jax
import jax.numpy as jnp
from jax import lax
from jax.experimental import pallas as pl
from jax.experimental.pallas import tpu as pltpu
from jax.experimental.pallas import tpu_sc as plsc

_NUM_SC_CORES = 2
_NUM_SC_SUBCORES = 16


def _sc_gather(gmf_emb, mlp_emb, i0, i1):
    B = i0.shape[0]
    D = gmf_emb.shape[1]
    nw = _NUM_SC_CORES * _NUM_SC_SUBCORES
    b_per_w = B // nw
    assert B % (8 * nw) == 0
    mesh = plsc.VectorSubcoreMesh(core_axis_name="c", subcore_axis_name="s")
    out_t = jax.ShapeDtypeStruct((B, D), jnp.float32)

    chunk = 128
    nbuf = 4
    n_chunks = b_per_w // chunk

    @functools.partial(
        pl.kernel,
        mesh=mesh,
        out_type=[out_t, out_t, out_t, out_t],
        scratch_types=[
            pltpu.VMEM((b_per_w,), jnp.int32),
            pltpu.VMEM((b_per_w,), jnp.int32),
        ] + [pltpu.VMEM((chunk, D), jnp.float32) for _ in range(nbuf)]
          + [pltpu.SemaphoreType.DMA for _ in range(2 * nbuf)],
    )
    def gather_kernel(gmf_hbm, mlp_hbm, i0_hbm, i1_hbm,
                      g0_hbm, g1_hbm, m0_hbm, m1_hbm,
                      idx0_v, idx1_v, *bufs_and_sems):
        bufs = bufs_and_sems[:nbuf]
        g_sems = bufs_and_sems[nbuf:2 * nbuf]
        w_sems = bufs_and_sems[2 * nbuf:]
        wid = lax.axis_index("s") * _NUM_SC_CORES + lax.axis_index("c")
        base = wid * b_per_w
        pltpu.sync_copy(i0_hbm.at[pl.ds(base, b_per_w)], idx0_v)
        pltpu.sync_copy(i1_hbm.at[pl.ds(base, b_per_w)], idx1_v)

        items = []
        for tbl, idx_v, out in ((gmf_hbm, idx0_v, g0_hbm),
                                (gmf_hbm, idx1_v, g1_hbm),
                                (mlp_hbm, idx0_v, m0_hbm),
                                (mlp_hbm, idx1_v, m1_hbm)):
            for c in range(n_chunks):
                items.append((tbl, idx_v, out, c * chunk))
        n = len(items)

        def start_gather(t):
            tbl, idx_v, _, off = items[t]
            b = t % nbuf
            return pltpu.async_copy(tbl.at[idx_v.at[pl.ds(off, chunk)]],
                                    bufs[b], g_sems[b])

        g_h = [None] * n
        w_h = [None] * n
        for t in range(min(nbuf, n)):
            g_h[t] = start_gather(t)
        for t in range(n):
            _, _, out, off = items[t]
            b = t % nbuf
            g_h[t].wait()
            w_h[t] = pltpu.async_copy(bufs[b],
                                      out.at[pl.ds(base + off, chunk)],
                                      w_sems[b])
            if t + nbuf < n:
                w_h[t].wait()
                g_h[t + nbuf] = start_gather(t + nbuf)
        for t in range(max(0, n - nbuf), n):
            w_h[t].wait()

    return gather_kernel(gmf_emb, mlp_emb, i0, i1)


def _tc_body(g0_r, g1_r, m0_r, m1_r, w1a_r, w1b_r, b1_r, w2_r, b2_r,
             w3_r, b3_r, wg_r, wm_r, bout_r, o_r):
    h = jnp.dot(m0_r[...], w1a_r[...], preferred_element_type=jnp.float32)
    h = h + jnp.dot(m1_r[...], w1b_r[...], preferred_element_type=jnp.float32)
    h = jnp.maximum(h + b1_r[...], 0.0)
    h = jnp.dot(h, w2_r[...], preferred_element_type=jnp.float32)
    h = jnp.maximum(h + b2_r[...], 0.0)
    h = jnp.dot(h, w3_r[...], preferred_element_type=jnp.float32)
    h = jnp.maximum(h + b3_r[...], 0.0)
    g = g0_r[...] * g1_r[...]
    dn = (((1,), (1,)), ((), ()))
    s = (lax.dot_general(wg_r[...], g, dn, preferred_element_type=jnp.float32)
         + lax.dot_general(wm_r[...], h, dn,
                           preferred_element_type=jnp.float32)
         + bout_r[...])
    o_r[...] = jax.nn.sigmoid(s)


def _tc_mlp(g0, g1, m0, m1, W1, b1, W2, b2, W3, b3, Wout, bout):
    B, D = g0.shape
    blk = 2048
    w1a = W1[:D]
    w1b = W1[D:]
    wg = Wout[:D].reshape(1, D)
    wm = Wout[D:].reshape(1, -1)
    grid = (B // blk,)

    def batch_spec():
        return pl.BlockSpec((blk, D), lambda i: (i, 0))

    def full_spec(shape):
        return pl.BlockSpec(shape, lambda i: tuple(0 for _ in shape))

    return pl.pallas_call(
        _tc_body,
        grid=grid,
        in_specs=[
            batch_spec(), batch_spec(), batch_spec(), batch_spec(),
            full_spec(w1a.shape), full_spec(w1b.shape),
            full_spec((1, b1.shape[0])),
            full_spec(W2.shape), full_spec((1, b2.shape[0])),
            full_spec(W3.shape), full_spec((1, b3.shape[0])),
            full_spec(wg.shape), full_spec(wm.shape),
            full_spec((1, 1)),
        ],
        out_specs=pl.BlockSpec((1, blk), lambda i: (0, i)),
        out_shape=jax.ShapeDtypeStruct((1, B), jnp.float32),
        compiler_params=pltpu.CompilerParams(
            dimension_semantics=("parallel",),
        ),
    )(g0, g1, m0, m1, w1a, w1b, b1.reshape(1, -1), W2, b2.reshape(1, -1),
      W3, b3.reshape(1, -1), wg, wm, bout.reshape(1, 1)).reshape(B, 1)


def kernel(x, gmf_emb, mlp_emb, W1, b1, W2, b2, W3, b3, Wout, bout):
    B = x.shape[0]
    i0 = x[:, 0]
    i1 = x[:, 1]
    n_rounds = 2
    h = B // n_rounds
    outs = []
    for r in range(n_rounds):
        sl = slice(r * h, (r + 1) * h)
        g0, g1, m0, m1 = _sc_gather(gmf_emb, mlp_emb, i0[sl], i1[sl])
        outs.append(_tc_mlp(g0, g1, m0, m1, W1, b1, W2, b2, W3, b3,
                            Wout, bout))
    return jnp.concatenate(outs, axis=0)


# speedup vs baseline: 1.0268x; 1.0268x over previous
import functools

import jax
import jax.numpy as jnp
from jax import lax
from jax.experimental import pallas as pl
from jax.experimental.pallas import tpu as pltpu
from jax.experimental.pallas import tpu_sc as plsc

_NUM_SC_CORES = 2
_NUM_SC_SUBCORES = 16
_LANES = 16


def _sc_gather(gmf_emb, mlp_emb, i0, i1):
    B = i0.shape[0]
    D = gmf_emb.shape[1]
    nw = _NUM_SC_CORES * _NUM_SC_SUBCORES
    b_per_w = B // nw
    assert B % (8 * nw) == 0
    chunk = 64
    n_chunks = b_per_w // chunk
    assert n_chunks >= 2 and b_per_w % chunk == 0
    mesh = plsc.VectorSubcoreMesh(core_axis_name="c", subcore_axis_name="s")
    out_t = jax.ShapeDtypeStruct((B, D), jnp.float32)

    buf_t = pltpu.VMEM((chunk, D), jnp.float32)
    n_bufs = 10

    @functools.partial(
        pl.kernel,
        mesh=mesh,
        out_type=[out_t, out_t, out_t],
        scratch_types=[
            pltpu.VMEM((b_per_w,), jnp.int32),
            pltpu.VMEM((b_per_w,), jnp.int32),
        ] + [buf_t] * n_bufs + [pltpu.SemaphoreType.DMA] * (2 * n_bufs),
    )
    def gather_kernel(gmf_hbm, mlp_hbm, i0_hbm, i1_hbm,
                      p_hbm, m0_hbm, m1_hbm,
                      idx0_v, idx1_v, *rest):
        bufs = rest[:n_bufs]
        g_sems = rest[n_bufs:2 * n_bufs]
        w_sems = rest[2 * n_bufs:]
        names = ("g0", "g1", "m0", "m1", "p")

        def bi(s, q):
            return names.index(s) * 2 + q

        wid = lax.axis_index("s") * _NUM_SC_CORES + lax.axis_index("c")
        base = wid * b_per_w
        pltpu.sync_copy(i0_hbm.at[pl.ds(base, b_per_w)], idx0_v)
        pltpu.sync_copy(i1_hbm.at[pl.ds(base, b_per_w)], idx1_v)

        gh = {}
        wh = {}

        def start_gathers(c):
            q = c % 2
            off = c * chunk
            s0 = idx0_v.at[pl.ds(off, chunk)]
            s1 = idx1_v.at[pl.ds(off, chunk)]
            for s, tbl, idx in (("g0", gmf_hbm, s0), ("g1", gmf_hbm, s1),
                                ("m0", mlp_hbm, s0), ("m1", mlp_hbm, s1)):
                b = bi(s, q)
                gh[(s, c)] = pltpu.async_copy(tbl.at[idx], bufs[b],
                                              g_sems[b])

        start_gathers(0)
        start_gathers(1)
        for c in range(n_chunks):
            q = c % 2
            off = c * chunk
            osl = pl.ds(base + off, chunk)
            gh[("g0", c)].wait()
            gh[("g1", c)].wait()
            pbuf = bufs[bi("p", q)]
            b0 = bufs[bi("g0", q)]
            b1 = bufs[bi("g1", q)]

            @pl.loop(0, chunk)
            def _(r):
                for l in range(0, D, _LANES):
                    slc = (pl.ds(r, 1), pl.ds(l, _LANES))
                    pbuf.at[slc][...] = b0.at[slc][...] * b1.at[slc][...]

            wh[("p", c)] = pltpu.async_copy(pbuf, p_hbm.at[osl],
                                            w_sems[bi("p", q)])
            gh[("m0", c)].wait()
            wh[("m0", c)] = pltpu.async_copy(bufs[bi("m0", q)],
                                             m0_hbm.at[osl],
                                             w_sems[bi("m0", q)])
            gh[("m1", c)].wait()
            wh[("m1", c)] = pltpu.async_copy(bufs[bi("m1", q)],
                                             m1_hbm.at[osl],
                                             w_sems[bi("m1", q)])
            if c + 2 < n_chunks:
                for s in ("p", "m0", "m1"):
                    wh[(s, c)].wait()
                start_gathers(c + 2)
        for c in (n_chunks - 2, n_chunks - 1):
            for s in ("p", "m0", "m1"):
                wh[(s, c)].wait()

    return gather_kernel(gmf_emb, mlp_emb, i0, i1)


def _tc_body(p_r, m0_r, m1_r, w1a_r, w1b_r, b1_r, w2_r, b2_r,
             w3_r, b3_r, wg_r, wm_r, bout_r, o_r):
    h = jnp.dot(m0_r[...], w1a_r[...], preferred_element_type=jnp.float32)
    h = h + jnp.dot(m1_r[...], w1b_r[...], preferred_element_type=jnp.float32)
    h = jnp.maximum(h + b1_r[...], 0.0)
    h = jnp.dot(h, w2_r[...], preferred_element_type=jnp.float32)
    h = jnp.maximum(h + b2_r[...], 0.0)
    h = jnp.dot(h, w3_r[...], preferred_element_type=jnp.float32)
    h = jnp.maximum(h + b3_r[...], 0.0)
    dn = (((1,), (1,)), ((), ()))
    s = (lax.dot_general(wg_r[...], p_r[...], dn,
                         preferred_element_type=jnp.float32)
         + lax.dot_general(wm_r[...], h, dn,
                           preferred_element_type=jnp.float32)
         + bout_r[...])
    o_r[...] = jax.nn.sigmoid(s)


def _tc_mlp(p, m0, m1, W1, b1, W2, b2, W3, b3, Wout, bout):
    B, D = p.shape
    blk = 2048
    w1a = W1[:D]
    w1b = W1[D:]
    wg = Wout[:D].reshape(1, D)
    wm = Wout[D:].reshape(1, -1)
    grid = (B // blk,)

    def batch_spec():
        return pl.BlockSpec((blk, D), lambda i: (i, 0))

    def full_spec(shape):
        return pl.BlockSpec(shape, lambda i: tuple(0 for _ in shape))

    return pl.pallas_call(
        _tc_body,
        grid=grid,
        in_specs=[
            batch_spec(), batch_spec(), batch_spec(),
            full_spec(w1a.shape), full_spec(w1b.shape),
            full_spec((1, b1.shape[0])),
            full_spec(W2.shape), full_spec((1, b2.shape[0])),
            full_spec(W3.shape), full_spec((1, b3.shape[0])),
            full_spec(wg.shape), full_spec(wm.shape),
            full_spec((1, 1)),
        ],
        out_specs=pl.BlockSpec((1, blk), lambda i: (0, i)),
        out_shape=jax.ShapeDtypeStruct((1, B), jnp.float32),
        compiler_params=pltpu.CompilerParams(
            dimension_semantics=("parallel",),
        ),
    )(p, m0, m1, w1a, w1b, b1.reshape(1, -1), W2, b2.reshape(1, -1),
      W3, b3.reshape(1, -1), wg, wm, bout.reshape(1, 1)).reshape(B, 1)


def kernel(x, gmf_emb, mlp_emb, W1, b1, W2, b2, W3, b3, Wout, bout):
    B = x.shape[0]
    i0 = x[:, 0]
    i1 = x[:, 1]
    n_rounds = 2
    h = B // n_rounds
    outs = []
    for r in range(n_rounds):
        sl = slice(r * h, (r + 1) * h)
        p, m0, m1 = _sc_gather(gmf_emb, mlp_emb, i0[sl], i1[sl])
        outs.append(_tc_mlp(p, m0, m1, W1, b1, W2, b2, W3, b3, Wout, bout))
    return jnp.concatenate(outs, axis=0)


# speedup vs baseline: 1.0459x; 1.0185x over previous
import functools

import jax
import jax.numpy as jnp
from jax import lax
from jax.experimental import pallas as pl
from jax.experimental.pallas import tpu as pltpu
from jax.experimental.pallas import tpu_sc as plsc

_NUM_SC_CORES = 2
_NUM_SC_SUBCORES = 16
_LANES = 16


def _sc_gather(gmf_emb, mlp_emb, i0, i1):
    B = i0.shape[0]
    D = gmf_emb.shape[1]
    nw = _NUM_SC_CORES * _NUM_SC_SUBCORES
    b_per_w = B // nw
    assert B % (8 * nw) == 0
    chunk = 64
    n_par = 3
    n_chunks = b_per_w // chunk
    assert n_chunks >= 2 and b_per_w % chunk == 0
    mesh = plsc.VectorSubcoreMesh(core_axis_name="c", subcore_axis_name="s")
    out_t = jax.ShapeDtypeStruct((B, D), jnp.float32)

    buf_t = pltpu.VMEM((chunk, D), jnp.float32)
    n_bufs = 5 * n_par

    @functools.partial(
        pl.kernel,
        mesh=mesh,
        out_type=[out_t, out_t, out_t],
        scratch_types=[
            pltpu.VMEM((b_per_w,), jnp.int32),
            pltpu.VMEM((b_per_w,), jnp.int32),
        ] + [buf_t] * n_bufs + [pltpu.SemaphoreType.DMA] * (2 * n_bufs),
    )
    def gather_kernel(gmf_hbm, mlp_hbm, i0_hbm, i1_hbm,
                      p_hbm, m0_hbm, m1_hbm,
                      idx0_v, idx1_v, *rest):
        bufs = rest[:n_bufs]
        g_sems = rest[n_bufs:2 * n_bufs]
        w_sems = rest[2 * n_bufs:]
        names = ("g0", "g1", "m0", "m1", "p")

        def bi(s, q):
            return names.index(s) * n_par + q

        wid = lax.axis_index("s") * _NUM_SC_CORES + lax.axis_index("c")
        base = wid * b_per_w
        pltpu.sync_copy(i0_hbm.at[pl.ds(base, b_per_w)], idx0_v)
        pltpu.sync_copy(i1_hbm.at[pl.ds(base, b_per_w)], idx1_v)

        gh = {}
        wh = {}

        def start_gathers(c):
            q = c % n_par
            off = c * chunk
            s0 = idx0_v.at[pl.ds(off, chunk)]
            s1 = idx1_v.at[pl.ds(off, chunk)]
            for s, tbl, idx in (("g0", gmf_hbm, s0), ("g1", gmf_hbm, s1),
                                ("m0", mlp_hbm, s0), ("m1", mlp_hbm, s1)):
                b = bi(s, q)
                gh[(s, c)] = pltpu.async_copy(tbl.at[idx], bufs[b],
                                              g_sems[b])

        for c in range(min(n_par, n_chunks)):
            start_gathers(c)
        for c in range(n_chunks):
            q = c % n_par
            off = c * chunk
            osl = pl.ds(base + off, chunk)
            gh[("g0", c)].wait()
            gh[("g1", c)].wait()
            pbuf = bufs[bi("p", q)]
            b0 = bufs[bi("g0", q)]
            b1 = bufs[bi("g1", q)]

            @pl.loop(0, chunk)
            def _(r):
                for l in range(0, D, _LANES):
                    slc = (pl.ds(r, 1), pl.ds(l, _LANES))
                    pbuf.at[slc][...] = b0.at[slc][...] * b1.at[slc][...]

            wh[("p", c)] = pltpu.async_copy(pbuf, p_hbm.at[osl],
                                            w_sems[bi("p", q)])
            gh[("m0", c)].wait()
            wh[("m0", c)] = pltpu.async_copy(bufs[bi("m0", q)],
                                             m0_hbm.at[osl],
                                             w_sems[bi("m0", q)])
            gh[("m1", c)].wait()
            wh[("m1", c)] = pltpu.async_copy(bufs[bi("m1", q)],
                                             m1_hbm.at[osl],
                                             w_sems[bi("m1", q)])
            if c + n_par < n_chunks:
                for s in ("p", "m0", "m1"):
                    wh[(s, c)].wait()
                start_gathers(c + n_par)
        for c in range(max(0, n_chunks - n_par), n_chunks):
            for s in ("p", "m0", "m1"):
                wh[(s, c)].wait()

    return gather_kernel(gmf_emb, mlp_emb, i0, i1)


def _tc_body(p_r, m0_r, m1_r, w1a_r, w1b_r, b1_r, w2_r, b2_r,
             w3_r, b3_r, wg_r, wm_r, bout_r, o_r):
    bf = jnp.bfloat16

    def mm(a, b):
        return jnp.dot(a.astype(bf), b.astype(bf),
                       preferred_element_type=jnp.float32)

    h = mm(m0_r[...], w1a_r[...]) + mm(m1_r[...], w1b_r[...])
    h = jnp.maximum(h + b1_r[...], 0.0)
    h = jnp.maximum(mm(h, w2_r[...]) + b2_r[...], 0.0)
    h = jnp.maximum(mm(h, w3_r[...]) + b3_r[...], 0.0)
    dn = (((1,), (1,)), ((), ()))
    s = (lax.dot_general(wg_r[...], p_r[...], dn,
                         preferred_element_type=jnp.float32)
         + lax.dot_general(wm_r[...], h, dn,
                           preferred_element_type=jnp.float32)
         + bout_r[...])
    o_r[...] = jax.nn.sigmoid(s)


def _tc_mlp(p, m0, m1, W1, b1, W2, b2, W3, b3, Wout, bout):
    B, D = p.shape
    blk = 2048
    w1a = W1[:D]
    w1b = W1[D:]
    wg = Wout[:D].reshape(1, D)
    wm = Wout[D:].reshape(1, -1)
    grid = (B // blk,)

    def batch_spec():
        return pl.BlockSpec((blk, D), lambda i: (i, 0))

    def full_spec(shape):
        return pl.BlockSpec(shape, lambda i: tuple(0 for _ in shape))

    return pl.pallas_call(
        _tc_body,
        grid=grid,
        in_specs=[
            batch_spec(), batch_spec(), batch_spec(),
            full_spec(w1a.shape), full_spec(w1b.shape),
            full_spec((1, b1.shape[0])),
            full_spec(W2.shape), full_spec((1, b2.shape[0])),
            full_spec(W3.shape), full_spec((1, b3.shape[0])),
            full_spec(wg.shape), full_spec(wm.shape),
            full_spec((1, 1)),
        ],
        out_specs=pl.BlockSpec((1, blk), lambda i: (0, i)),
        out_shape=jax.ShapeDtypeStruct((1, B), jnp.float32),
        compiler_params=pltpu.CompilerParams(
            dimension_semantics=("parallel",),
        ),
    )(p, m0, m1, w1a, w1b, b1.reshape(1, -1), W2, b2.reshape(1, -1),
      W3, b3.reshape(1, -1), wg, wm, bout.reshape(1, 1)).reshape(B, 1)


def kernel(x, gmf_emb, mlp_emb, W1, b1, W2, b2, W3, b3, Wout, bout):
    B = x.shape[0]
    i0 = x[:, 0]
    i1 = x[:, 1]
    n_rounds = 2
    h = B // n_rounds
    outs = []
    for r in range(n_rounds):
        sl = slice(r * h, (r + 1) * h)
        p, m0, m1 = _sc_gather(gmf_emb, mlp_emb, i0[sl], i1[sl])
        outs.append(_tc_mlp(p, m0, m1, W1, b1, W2, b2, W3, b3, Wout, bout))
    return jnp.concatenate(outs, axis=0)
